# Initial kernel scaffold; baseline (speedup 1.0000x reference)
import jax
import jax.numpy as jnp
from jax.experimental import pallas as pl


def kernel(z_fast_real, z_fast_imag, z_slow_real, z_slow_imag, cb_syn, cb_sem):
    raise NotImplementedError("write your pallas kernel here")


# speedup vs baseline: 1.1812x; 1.1812x over previous
import jax
import jax.numpy as jnp
from jax.experimental import pallas as pl
from jax.experimental.pallas import tpu as pltpu
from jax.experimental.pallas import tpu_sc as plsc

_N = 8192
_DIM = 512
_NCB = 4096
_LAT = 256
_BLK = 1024
_GW = 128


def _dist_body(z_ref, cb_ref, idx_ref, dsum_ref, cbsq_ref):
    step = pl.program_id(0)

    @pl.when(step == 0)
    def _():
        cb = cb_ref[...]
        cbsq_ref[0, :] = jnp.sum(cb * cb, axis=1)

    z = z_ref[...]
    zsq = jnp.sum(z * z, axis=1)
    mm = jax.lax.dot_general(
        z, cb_ref[...], (((1,), (1,)), ((), ())),
        preferred_element_type=jnp.float32)
    d = (zsq[:, None] + cbsq_ref[0:1, :]) - 2.0 * mm
    m = jnp.min(d, axis=1)
    iota = jax.lax.broadcasted_iota(jnp.int32, d.shape, 1)
    hit = jnp.where(d == m[:, None], iota, _NCB)
    idx_ref[...] = jnp.min(hit, axis=1).astype(jnp.int32)
    bsum = jnp.sum(m)

    @pl.when(step == 0)
    def _():
        dsum_ref[0, 0] = bsum

    @pl.when(step != 0)
    def _():
        dsum_ref[0, 0] += bsum


def _vq_dist(z, cb):
    return pl.pallas_call(
        _dist_body,
        grid=(_N // _BLK,),
        in_specs=[
            pl.BlockSpec((_BLK, _DIM), lambda i: (i, 0)),
            pl.BlockSpec((_NCB, _DIM), lambda i: (0, 0)),
        ],
        out_specs=[
            pl.BlockSpec((_BLK,), lambda i: (i,)),
            pl.BlockSpec(memory_space=pltpu.SMEM),
        ],
        out_shape=[
            jax.ShapeDtypeStruct((_N,), jnp.int32),
            jax.ShapeDtypeStruct((1, 1), jnp.float32),
        ],
        scratch_shapes=[pltpu.VMEM((1, _NCB), jnp.float32)],
    )(z, cb)


def _sc_gather(cb, idx):
    mesh = plsc.VectorSubcoreMesh(core_axis_name="c", subcore_axis_name="s")
    idx2 = idx.reshape((1, _N))
    half_t = jax.ShapeDtypeStruct((_N, _LAT), cb.dtype)

    @pl.kernel(out_type=(half_t, half_t), mesh=mesh)
    def kern(cb_hbm, i_hbm, or_hbm, oi_hbm):
        def run(col0, o_hbm):
            def body(i_vmem, o_vmem):
                pltpu.sync_copy(
                    cb_hbm.at[i_vmem.at[0], pl.ds(col0, _LAT)], o_vmem)

            pltpu.emit_pipeline(
                body,
                grid=(_N // _GW,),
                in_specs=[pl.BlockSpec((1, _GW), lambda i: (0, i))],
                out_specs=[pl.BlockSpec((_GW, _LAT), lambda i: (i, 0))],
                core_axis_name=("c", "s"),
                dimension_semantics=(pltpu.PARALLEL,),
            )(i_hbm, o_hbm)

        run(0, or_hbm)
        run(_LAT, oi_hbm)

    return kern(cb, idx2)


def kernel(z_fast_real, z_fast_imag, z_slow_real, z_slow_imag, cb_syn, cb_sem):
    zf = jnp.concatenate([z_fast_real, z_fast_imag], axis=-1)
    zs = jnp.concatenate([z_slow_real, z_slow_imag], axis=-1)
    idx_syn, dsum_syn = _vq_dist(zf, cb_syn)
    idx_sem, dsum_sem = _vq_dist(zs, cb_sem)
    zq_syn_re, zq_syn_im = _sc_gather(cb_syn, idx_syn)
    zq_sem_re, zq_sem_im = _sc_gather(cb_sem, idx_sem)
    loss = (1.25 / (_N * _DIM)) * (dsum_syn[0, 0] + dsum_sem[0, 0])
    zq_syn_c = jax.lax.complex(zq_syn_re, zq_syn_im)
    zq_sem_c = jax.lax.complex(zq_sem_re, zq_sem_im)
    return (zq_syn_c, zq_sem_c, loss, idx_syn, idx_sem)


# speedup vs baseline: 1.2778x; 1.0818x over previous
import jax
import jax.numpy as jnp
from jax.experimental import pallas as pl
from jax.experimental.pallas import tpu as pltpu
from jax.experimental.pallas import tpu_sc as plsc

_N = 8192
_DIM = 512
_NCB = 4096
_LAT = 256
_BLK = 1024
_GW = 128


def _dist_body(z_ref, cb_ref, idx_ref, cbsq_ref):
    step = pl.program_id(0)

    @pl.when(step == 0)
    def _():
        cb = cb_ref[...]
        cbsq_ref[0, :] = jnp.sum(cb * cb, axis=1)

    z = z_ref[...]
    zsq = jnp.sum(z * z, axis=1)
    mm = jax.lax.dot_general(
        z, cb_ref[...], (((1,), (1,)), ((), ())),
        preferred_element_type=jnp.float32)
    d = (zsq[:, None] + cbsq_ref[0:1, :]) - 2.0 * mm
    idx_ref[...] = jnp.argmin(d, axis=1).astype(jnp.int32)


def _vq_dist(z, cb):
    return pl.pallas_call(
        _dist_body,
        grid=(_N // _BLK,),
        in_specs=[
            pl.BlockSpec((_BLK, _DIM), lambda i: (i, 0)),
            pl.BlockSpec((_NCB, _DIM), lambda i: (0, 0)),
        ],
        out_specs=pl.BlockSpec((_BLK,), lambda i: (i,)),
        out_shape=jax.ShapeDtypeStruct((_N,), jnp.int32),
        scratch_shapes=[pltpu.VMEM((1, _NCB), jnp.float32)],
    )(z, cb)


def _loss_body(zf_ref, sre_ref, sim_ref, zs_ref, mre_ref, mim_ref, out_ref):
    step = pl.program_id(0)
    ds = (zf_ref[:, :_LAT] - sre_ref[...]) ** 2
    ds += (zf_ref[:, _LAT:] - sim_ref[...]) ** 2
    dm = (zs_ref[:, :_LAT] - mre_ref[...]) ** 2
    dm += (zs_ref[:, _LAT:] - mim_ref[...]) ** 2
    bsum = jnp.sum(ds) + jnp.sum(dm)

    @pl.when(step == 0)
    def _():
        out_ref[0, 0] = bsum

    @pl.when(step != 0)
    def _():
        out_ref[0, 0] += bsum


def _loss_sum(zf, sre, sim, zs, mre, mim):
    full = pl.BlockSpec((_BLK, _DIM), lambda i: (i, 0))
    half = pl.BlockSpec((_BLK, _LAT), lambda i: (i, 0))
    return pl.pallas_call(
        _loss_body,
        grid=(_N // _BLK,),
        in_specs=[full, half, half, full, half, half],
        out_specs=pl.BlockSpec(memory_space=pltpu.SMEM),
        out_shape=jax.ShapeDtypeStruct((1, 1), jnp.float32),
    )(zf, sre, sim, zs, mre, mim)


def _sc_gather(cb, idx):
    mesh = plsc.VectorSubcoreMesh(core_axis_name="c", subcore_axis_name="s")
    idx2 = idx.reshape((1, _N))
    half_t = jax.ShapeDtypeStruct((_N, _LAT), cb.dtype)

    @pl.kernel(out_type=(half_t, half_t), mesh=mesh)
    def kern(cb_hbm, i_hbm, or_hbm, oi_hbm):
        def run(col0, o_hbm):
            def body(i_vmem, o_vmem):
                pltpu.sync_copy(
                    cb_hbm.at[i_vmem.at[0], pl.ds(col0, _LAT)], o_vmem)

            pltpu.emit_pipeline(
                body,
                grid=(_N // _GW,),
                in_specs=[pl.BlockSpec((1, _GW), lambda i: (0, i))],
                out_specs=[pl.BlockSpec((_GW, _LAT), lambda i: (i, 0))],
                core_axis_name=("c", "s"),
                dimension_semantics=(pltpu.PARALLEL,),
            )(i_hbm, o_hbm)

        run(0, or_hbm)
        run(_LAT, oi_hbm)

    return kern(cb, idx2)


def kernel(z_fast_real, z_fast_imag, z_slow_real, z_slow_imag, cb_syn, cb_sem):
    zf = jnp.concatenate([z_fast_real, z_fast_imag], axis=-1)
    zs = jnp.concatenate([z_slow_real, z_slow_imag], axis=-1)
    idx_syn = _vq_dist(zf, cb_syn)
    idx_sem = _vq_dist(zs, cb_sem)
    zq_syn_re, zq_syn_im = _sc_gather(cb_syn, idx_syn)
    zq_sem_re, zq_sem_im = _sc_gather(cb_sem, idx_sem)
    dsum = _loss_sum(zf, zq_syn_re, zq_syn_im, zs, zq_sem_re, zq_sem_im)
    loss = (1.25 / (_N * _DIM)) * dsum[0, 0]
    zq_syn_c = jax.lax.complex(zq_syn_re, zq_syn_im)
    zq_sem_c = jax.lax.complex(zq_sem_re, zq_sem_im)
    return (zq_syn_c, zq_sem_c, loss, idx_syn, idx_sem)


# speedup vs baseline: 1.3639x; 1.0674x over previous
import jax
import jax.numpy as jnp
from jax.experimental import pallas as pl
from jax.experimental.pallas import tpu as pltpu
from jax.experimental.pallas import tpu_sc as plsc

_N = 8192
_DIM = 512
_NCB = 4096
_LAT = 256
_BLK = 1024
_GW = 128


def _dist_body(zr_ref, zi_ref, cb_ref, idx_ref, cbsq_ref):
    step = pl.program_id(0)

    @pl.when(step == 0)
    def _():
        cb = cb_ref[...]
        cbsq_ref[0, :] = jnp.sum(cb * cb, axis=1)

    z = jnp.concatenate([zr_ref[...], zi_ref[...]], axis=1)
    zsq = jnp.sum(z * z, axis=1)
    mm2 = jax.lax.dot_general(
        2.0 * z, cb_ref[...], (((1,), (1,)), ((), ())),
        preferred_element_type=jnp.float32)
    d = (zsq[:, None] + cbsq_ref[0:1, :]) - mm2
    idx_ref[...] = jnp.argmin(d, axis=1).astype(jnp.int32)


def _vq_dist(zr, zi, cb):
    half = pl.BlockSpec((_BLK, _LAT), lambda i: (i, 0))
    return pl.pallas_call(
        _dist_body,
        grid=(_N // _BLK,),
        in_specs=[half, half, pl.BlockSpec((_NCB, _DIM), lambda i: (0, 0))],
        out_specs=pl.BlockSpec((_BLK,), lambda i: (i,)),
        out_shape=jax.ShapeDtypeStruct((_N,), jnp.int32),
        scratch_shapes=[pltpu.VMEM((1, _NCB), jnp.float32)],
    )(zr, zi, cb)


def _loss_body(fr_ref, fi_ref, sre_ref, sim_ref,
               sr_ref, si_ref, mre_ref, mim_ref, out_ref):
    step = pl.program_id(0)
    acc = (fr_ref[...] - sre_ref[...]) ** 2
    acc += (fi_ref[...] - sim_ref[...]) ** 2
    acc += (sr_ref[...] - mre_ref[...]) ** 2
    acc += (si_ref[...] - mim_ref[...]) ** 2
    bsum = jnp.sum(acc)

    @pl.when(step == 0)
    def _():
        out_ref[0, 0] = bsum

    @pl.when(step != 0)
    def _():
        out_ref[0, 0] += bsum


def _loss_sum(fr, fi, sre, sim, sr, si, mre, mim):
    half = pl.BlockSpec((_BLK, _LAT), lambda i: (i, 0))
    return pl.pallas_call(
        _loss_body,
        grid=(_N // _BLK,),
        in_specs=[half] * 8,
        out_specs=pl.BlockSpec(memory_space=pltpu.SMEM),
        out_shape=jax.ShapeDtypeStruct((1, 1), jnp.float32),
    )(fr, fi, sre, sim, sr, si, mre, mim)


def _sc_gather(cb, idx):
    mesh = plsc.VectorSubcoreMesh(core_axis_name="c", subcore_axis_name="s")
    idx2 = idx.reshape((1, _N))
    half_t = jax.ShapeDtypeStruct((_N, _LAT), cb.dtype)

    @pl.kernel(out_type=(half_t, half_t), mesh=mesh)
    def kern(cb_hbm, i_hbm, or_hbm, oi_hbm):
        def run(col0, o_hbm):
            def body(i_vmem, o_vmem):
                pltpu.sync_copy(
                    cb_hbm.at[i_vmem.at[0], pl.ds(col0, _LAT)], o_vmem)

            pltpu.emit_pipeline(
                body,
                grid=(_N // _GW,),
                in_specs=[pl.BlockSpec((1, _GW), lambda i: (0, i))],
                out_specs=[pl.BlockSpec((_GW, _LAT), lambda i: (i, 0))],
                core_axis_name=("c", "s"),
                dimension_semantics=(pltpu.PARALLEL,),
            )(i_hbm, o_hbm)

        run(0, or_hbm)
        run(_LAT, oi_hbm)

    return kern(cb, idx2)


def kernel(z_fast_real, z_fast_imag, z_slow_real, z_slow_imag, cb_syn, cb_sem):
    idx_syn = _vq_dist(z_fast_real, z_fast_imag, cb_syn)
    idx_sem = _vq_dist(z_slow_real, z_slow_imag, cb_sem)
    zq_syn_re, zq_syn_im = _sc_gather(cb_syn, idx_syn)
    zq_sem_re, zq_sem_im = _sc_gather(cb_sem, idx_sem)
    dsum = _loss_sum(z_fast_real, z_fast_imag, zq_syn_re, zq_syn_im,
                     z_slow_real, z_slow_imag, zq_sem_re, zq_sem_im)
    loss = (1.25 / (_N * _DIM)) * dsum[0, 0]
    zq_syn_c = jax.lax.complex(zq_syn_re, zq_syn_im)
    zq_sem_c = jax.lax.complex(zq_sem_re, zq_sem_im)
    return (zq_syn_c, zq_sem_c, loss, idx_syn, idx_sem)
